# baseline (speedup 1.0000x reference)
import functools

import jax
import jax.numpy as jnp
from jax import lax
from jax.experimental import pallas as pl
from jax.experimental.pallas import tpu as pltpu
from jax.experimental.pallas import tpu_sc as plsc

CROP_H = 192
CROP_W = 192
NUM_CROPS = 4
NW = 32


def kernel(inputs):
    B, C, H, W = inputs.shape
    max_y = H - CROP_H
    max_x = W - CROP_W
    key = jax.random.key(42)
    ky, kx = jax.random.split(key)
    rand_y = (jax.random.uniform(ky, (B, NUM_CROPS)) * max_y).astype(jnp.int32)
    rand_x = (jax.random.uniform(kx, (B, NUM_CROPS)) * max_x).astype(jnp.int32)

    G = B * C
    IPW = G // NW
    KPW = IPW * NUM_CROPS

    g = jnp.arange(G, dtype=jnp.int32)
    y_meta = rand_y[g // C]
    x_meta = rand_x[g // C]

    def bcast(v):
        return jnp.broadcast_to(
            v.reshape(NW, KPW, 1), (NW, KPW, 16)).astype(jnp.int32)

    zero = lax.optimization_barrier(jnp.float32(0.0))
    in_flat = (inputs + zero).reshape(G * H * W)
    T = B * NUM_CROPS * C
    CROP_SZ = CROP_H * CROP_W

    mesh = plsc.VectorSubcoreMesh(core_axis_name="c", subcore_axis_name="s")

    @functools.partial(
        pl.kernel,
        mesh=mesh,
        compiler_params=pltpu.CompilerParams(
            use_tc_tiling_on_sc=False, needs_layout_passes=False),
        out_type=jax.ShapeDtypeStruct((T * CROP_SZ,), jnp.float32),
        scratch_types=[
            pltpu.VMEM((KPW, 16), jnp.int32),
            pltpu.VMEM((KPW, 16), jnp.int32),
            pltpu.VMEM((H * W,), jnp.float32),
            pltpu.VMEM((CROP_SZ,), jnp.float32),
            pltpu.VMEM((CROP_SZ,), jnp.float32),
            pltpu.SemaphoreType.DMA,
            pltpu.SemaphoreType.DMA,
            pltpu.SemaphoreType.DMA,
        ],
    )
    def crop_kernel(in_hbm, y_hbm, x_hbm, out_hbm, y_v, x_v, img_v,
                    crop0_v, crop1_v, sem0, sem1, isem):
        nc = 2
        wid = lax.axis_index("s") * nc + lax.axis_index("c")
        pltpu.sync_copy(y_hbm.at[wid], y_v)
        pltpu.sync_copy(x_hbm.at[wid], x_v)
        crops = (crop0_v, crop1_v)
        sems = (sem0, sem1)
        pending = [None, None]

        def img_load(m):
            return pltpu.async_copy(
                in_hbm.at[pl.ds((wid * IPW + m) * (H * W), H * W)],
                img_v, isem)

        img_pending = img_load(0)
        for m in range(IPW):
            img_pending.wait()
            for n in range(NUM_CROPS):
                k = m * NUM_CROPS + n
                buf = k % 2
                y = jnp.max(y_v[k, :])
                x = jnp.max(x_v[k, :])
                if pending[buf] is not None:
                    pending[buf].wait()
                crop_v = crops[buf]
                base = y * W + x

                @plsc.parallel_loop(0, CROP_H, unroll=8)
                def row_body(i):
                    for j in range(CROP_W // 16):
                        crop_v[pl.ds(i * CROP_W + 16 * j, 16)] = (
                            img_v[pl.ds(base + i * W + 16 * j, 16)])

                if n == NUM_CROPS - 1 and m < IPW - 1:
                    img_pending = img_load(m + 1)
                t = wid * KPW + (m // C) * NUM_CROPS * C + n * C + m % C
                pending[buf] = pltpu.async_copy(
                    crop_v, out_hbm.at[pl.ds(t * CROP_SZ, CROP_SZ)],
                    sems[buf])
        pending[0].wait()
        pending[1].wait()

    out = crop_kernel(in_flat, bcast(y_meta), bcast(x_meta))
    return (out + zero).reshape(B * NUM_CROPS, C, CROP_H, CROP_W)


# speedup vs baseline: 1.2567x; 1.2567x over previous
import functools

import jax
import jax.numpy as jnp
from jax import lax
from jax.experimental import pallas as pl
from jax.experimental.pallas import tpu as pltpu
from jax.experimental.pallas import tpu_sc as plsc

CROP_H = 192
CROP_W = 192
NUM_CROPS = 4
NW = 32


def kernel(inputs):
    B, C, H, W = inputs.shape
    max_y = H - CROP_H
    max_x = W - CROP_W
    key = jax.random.key(42)
    ky, kx = jax.random.split(key)
    rand_y = (jax.random.uniform(ky, (B, NUM_CROPS)) * max_y).astype(jnp.int32)
    rand_x = (jax.random.uniform(kx, (B, NUM_CROPS)) * max_x).astype(jnp.int32)

    G = B * C
    IPW = G // NW
    KPW = IPW * NUM_CROPS

    g = jnp.arange(G, dtype=jnp.int32)
    y_meta = rand_y[g // C]
    x_meta = rand_x[g // C]

    def bcast(v):
        return jnp.broadcast_to(
            v.reshape(NW, KPW, 1), (NW, KPW, 16)).astype(jnp.int32)

    in_flat = inputs.reshape(G * H * W)
    T = B * NUM_CROPS * C
    CROP_SZ = CROP_H * CROP_W

    mesh = plsc.VectorSubcoreMesh(core_axis_name="c", subcore_axis_name="s")

    @functools.partial(
        pl.kernel,
        mesh=mesh,
        compiler_params=pltpu.CompilerParams(
            use_tc_tiling_on_sc=False, needs_layout_passes=False),
        out_type=jax.ShapeDtypeStruct((T * CROP_SZ,), jnp.float32),
        scratch_types=[
            pltpu.VMEM((KPW, 16), jnp.int32),
            pltpu.VMEM((KPW, 16), jnp.int32),
            pltpu.VMEM((H * W,), jnp.float32),
            pltpu.VMEM((CROP_SZ,), jnp.float32),
            pltpu.VMEM((CROP_SZ,), jnp.float32),
            pltpu.SemaphoreType.DMA,
            pltpu.SemaphoreType.DMA,
            pltpu.SemaphoreType.DMA,
        ],
    )
    def crop_kernel(in_hbm, y_hbm, x_hbm, out_hbm, y_v, x_v, img_v,
                    crop0_v, crop1_v, sem0, sem1, isem):
        nc = 2
        wid = lax.axis_index("s") * nc + lax.axis_index("c")
        pltpu.sync_copy(y_hbm.at[wid], y_v)
        pltpu.sync_copy(x_hbm.at[wid], x_v)
        crops = (crop0_v, crop1_v)
        sems = (sem0, sem1)
        pending = [None, None]

        def img_load(m):
            return pltpu.async_copy(
                in_hbm.at[pl.ds((wid * IPW + m) * (H * W), H * W)],
                img_v, isem)

        img_pending = img_load(0)
        for m in range(IPW):
            img_pending.wait()
            for n in range(NUM_CROPS):
                k = m * NUM_CROPS + n
                buf = k % 2
                y = jnp.max(y_v[k, :])
                x = jnp.max(x_v[k, :])
                if pending[buf] is not None:
                    pending[buf].wait()
                crop_v = crops[buf]
                base = y * W + x

                @plsc.parallel_loop(0, CROP_H, unroll=8)
                def row_body(i):
                    for j in range(CROP_W // 16):
                        crop_v[pl.ds(i * CROP_W + 16 * j, 16)] = (
                            img_v[pl.ds(base + i * W + 16 * j, 16)])

                if n == NUM_CROPS - 1 and m < IPW - 1:
                    img_pending = img_load(m + 1)
                t = wid * KPW + (m // C) * NUM_CROPS * C + n * C + m % C
                pending[buf] = pltpu.async_copy(
                    crop_v, out_hbm.at[pl.ds(t * CROP_SZ, CROP_SZ)],
                    sems[buf])
        pending[0].wait()
        pending[1].wait()

    out = crop_kernel(in_flat, bcast(y_meta), bcast(x_meta))
    return out.reshape(B * NUM_CROPS, C, CROP_H, CROP_W)
